# Initial kernel scaffold; baseline (speedup 1.0000x reference)
import jax
import jax.numpy as jnp
from jax.experimental import pallas as pl


def kernel(feat_d, feat_w, edge_dd, edge_dw, edge_wd, basis0, coeff0, bias0, basis1, coeff1, bias1, basis2, coeff2, bias2):
    raise NotImplementedError("write your pallas kernel here")


# speedup vs baseline: 1.3637x; 1.3637x over previous
import functools

import jax
import jax.numpy as jnp
from jax import lax
from jax.experimental import pallas as pl
from jax.experimental.pallas import tpu as pltpu
from jax.experimental.pallas import tpu_sc as plsc

ND = 50000
NW = 50000
NE = 200000
H = 128
DOUT = 16
HC = 32
NCH = H // HC

NC = 2
NS = 16
NTILES = NC * NS

NPAD = 50176
EB = 128
NBLK = 49
EPT = NBLK * EB
EPAD = NTILES * EPT

ROWS_PT = NPAD // NS
ZROWS = ROWS_PT // 8

_f32 = jnp.float32


def _sc_mesh():
    return plsc.VectorSubcoreMesh(core_axis_name="c", subcore_axis_name="s")


def _make_segsum(C, W):
    @functools.partial(
        pl.kernel,
        mesh=_sc_mesh(),
        compiler_params=pltpu.CompilerParams(use_tc_tiling_on_sc=False),
        out_type=jax.ShapeDtypeStruct((NC, C, NPAD, W), _f32),
        scratch_types=[
            pltpu.VMEM((EB,), jnp.int32),
            pltpu.VMEM((EB,), jnp.int32),
            pltpu.VMEM((EB, W), _f32),
            pltpu.VMEM((ZROWS, W), _f32),
            pltpu.VMEM_SHARED((NPAD, W), _f32),
            pltpu.SemaphoreType.DMA,
        ],
    )
    def segsum(table_hbm, sidx_hbm, didx_hbm, out_hbm,
               sidx_v, didx_v, rows_v, zeros_v, acc_sh, sem):
        cid = lax.axis_index("c")
        sid = lax.axis_index("s")

        @pl.loop(0, ZROWS)
        def _zinit(i):
            for j in range(W // 16):
                zeros_v[i, pl.ds(j * 16, 16)] = jnp.zeros((16,), _f32)

        for c in range(C):
            for z in range(ROWS_PT // ZROWS):
                pltpu.sync_copy(
                    zeros_v,
                    acc_sh.at[pl.ds(sid * ROWS_PT + z * ZROWS, ZROWS)])
            plsc.subcore_barrier()

            @pl.loop(0, NBLK)
            def _eblk(j):
                pltpu.sync_copy(sidx_hbm.at[c, cid, sid, j], sidx_v)
                pltpu.sync_copy(didx_hbm.at[cid, sid, j], didx_v)
                pltpu.async_copy(table_hbm.at[sidx_v], rows_v, sem).wait()
                pltpu.sync_copy(rows_v, acc_sh.at[didx_v], add=True)

            plsc.subcore_barrier()
            pltpu.sync_copy(
                acc_sh.at[pl.ds(sid * ROWS_PT, ROWS_PT)],
                out_hbm.at[cid, c, pl.ds(sid * ROWS_PT, ROWS_PT)])
            plsc.subcore_barrier()

    return segsum


def _make_deg():
    @functools.partial(
        pl.kernel,
        mesh=_sc_mesh(),
        compiler_params=pltpu.CompilerParams(use_tc_tiling_on_sc=False),
        out_type=jax.ShapeDtypeStruct((NC, NPAD, 16), _f32),
        scratch_types=[
            pltpu.VMEM((EB,), jnp.int32),
            pltpu.VMEM((EB, 16), _f32),
            pltpu.VMEM((ZROWS, 16), _f32),
            pltpu.VMEM_SHARED((NPAD, 16), _f32),
        ],
    )
    def deg(didx_hbm, out_hbm, didx_v, ones_v, zeros_v, acc_sh):
        cid = lax.axis_index("c")
        sid = lax.axis_index("s")

        @pl.loop(0, EB)
        def _oinit(i):
            ones_v[i, pl.ds(0, 16)] = jnp.ones((16,), _f32)

        @pl.loop(0, ZROWS)
        def _zinit(i):
            zeros_v[i, pl.ds(0, 16)] = jnp.zeros((16,), _f32)

        for z in range(ROWS_PT // ZROWS):
            pltpu.sync_copy(
                zeros_v, acc_sh.at[pl.ds(sid * ROWS_PT + z * ZROWS, ZROWS)])
        plsc.subcore_barrier()

        @pl.loop(0, NBLK)
        def _eblk(j):
            pltpu.sync_copy(didx_hbm.at[cid, sid, j], didx_v)
            pltpu.sync_copy(ones_v, acc_sh.at[didx_v], add=True)

        plsc.subcore_barrier()
        pltpu.sync_copy(
            acc_sh.at[pl.ds(sid * ROWS_PT, ROWS_PT)],
            out_hbm.at[cid, pl.ds(sid * ROWS_PT, ROWS_PT)])

    return deg


_R = 256


def _mean(a_ref, d_ref):
    a = a_ref[0] + a_ref[1]
    deg = d_ref[0, :, 0:1] + d_ref[1, :, 0:1]
    r = 1.0 / jnp.maximum(deg, 1.0)
    return a * r[None]


def _combine01_body(coeff_ref, basis_ref, bias_ref,
                    add_ref, awd_ref, adw_ref,
                    ddd_ref, dwd_ref, ddw_ref,
                    outd_ref, outw_ref, w_ref):
    i = pl.program_id(0)

    @pl.when(i == 0)
    def _():
        bs = basis_ref[...]
        for r in range(3):
            w_ref[r] = coeff_ref[r, 0] * bs[0] + coeff_ref[r, 1] * bs[1]

    xdd = _mean(add_ref, ddd_ref)
    xwd = _mean(awd_ref, dwd_ref)
    xdw = _mean(adw_ref, ddw_ref)
    w = w_ref[...]
    accd = jnp.zeros((_R, H), _f32)
    accw = jnp.zeros((_R, H), _f32)
    for c in range(NCH):
        ws = w[:, c * HC:(c + 1) * HC, :]
        accd = accd + jnp.dot(xdd[c], ws[0], preferred_element_type=_f32,
                              precision=lax.Precision.HIGHEST)
        accd = accd + jnp.dot(xwd[c], ws[2], preferred_element_type=_f32,
                              precision=lax.Precision.HIGHEST)
        accw = accw + jnp.dot(xdw[c], ws[1], preferred_element_type=_f32,
                              precision=lax.Precision.HIGHEST)
    hd = jnp.maximum(accd + bias_ref[...], 0.0)
    hw = jnp.maximum(accw + bias_ref[...], 0.0)
    for c in range(NCH):
        outd_ref[c] = hd[:, c * HC:(c + 1) * HC]
        outw_ref[c] = hw[:, c * HC:(c + 1) * HC]


def _combine01(coeff, basis, bias, add, awd, adw, degdd, degwd, degdw):
    agg_spec = pl.BlockSpec((NC, NCH, _R, HC), lambda i: (0, 0, i, 0))
    deg_spec = pl.BlockSpec((NC, _R, 16), lambda i: (0, i, 0))
    out_spec = pl.BlockSpec((NCH, _R, HC), lambda i: (0, i, 0))
    return pl.pallas_call(
        _combine01_body,
        grid=(NPAD // _R,),
        in_specs=[
            pl.BlockSpec(memory_space=pltpu.SMEM),
            pl.BlockSpec((2, H, H), lambda i: (0, 0, 0)),
            pl.BlockSpec((1, H), lambda i: (0, 0)),
            agg_spec, agg_spec, agg_spec,
            deg_spec, deg_spec, deg_spec,
        ],
        out_specs=[out_spec, out_spec],
        out_shape=[
            jax.ShapeDtypeStruct((NCH, NPAD, HC), _f32),
            jax.ShapeDtypeStruct((NCH, NPAD, HC), _f32),
        ],
        scratch_shapes=[pltpu.VMEM((3, H, H), _f32)],
    )(coeff, basis, bias, add, awd, adw, degdd, degwd, degdw)


def _premul_body(coeff_ref, basis_ref, hd_ref, hw_ref,
                 pdd_ref, pwd_ref, w_ref):
    i = pl.program_id(0)

    @pl.when(i == 0)
    def _():
        bs = basis_ref[...]
        w_ref[0] = coeff_ref[0, 0] * bs[0] + coeff_ref[0, 1] * bs[1]
        w_ref[1] = coeff_ref[2, 0] * bs[0] + coeff_ref[2, 1] * bs[1]

    w = w_ref[...]
    pdd = jnp.zeros((_R, DOUT), _f32)
    pwd = jnp.zeros((_R, DOUT), _f32)
    for c in range(NCH):
        ws = w[:, c * HC:(c + 1) * HC, :]
        pdd = pdd + jnp.dot(hd_ref[c], ws[0], preferred_element_type=_f32,
                              precision=lax.Precision.HIGHEST)
        pwd = pwd + jnp.dot(hw_ref[c], ws[1], preferred_element_type=_f32,
                              precision=lax.Precision.HIGHEST)
    pdd_ref[...] = pdd
    pwd_ref[...] = pwd


def _premul(coeff, basis, hd, hw):
    h_spec = pl.BlockSpec((NCH, _R, HC), lambda i: (0, i, 0))
    out_spec = pl.BlockSpec((_R, DOUT), lambda i: (i, 0))
    return pl.pallas_call(
        _premul_body,
        grid=(NPAD // _R,),
        in_specs=[
            pl.BlockSpec(memory_space=pltpu.SMEM),
            pl.BlockSpec((2, H, DOUT), lambda i: (0, 0, 0)),
            h_spec, h_spec,
        ],
        out_specs=[out_spec, out_spec],
        out_shape=[
            jax.ShapeDtypeStruct((NPAD, DOUT), _f32),
            jax.ShapeDtypeStruct((NPAD, DOUT), _f32),
        ],
        scratch_shapes=[pltpu.VMEM((2, H, DOUT), _f32)],
    )(coeff, basis, hd, hw)


def _final_body(bias_ref, add_ref, awd_ref, ddd_ref, dwd_ref, out_ref):
    xdd = _mean(add_ref, ddd_ref)
    xwd = _mean(awd_ref, dwd_ref)
    out_ref[...] = xdd[0] + xwd[0] + bias_ref[...]


def _final(bias, add, awd, degdd, degwd):
    agg_spec = pl.BlockSpec((NC, 1, _R, DOUT), lambda i: (0, 0, i, 0))
    deg_spec = pl.BlockSpec((NC, _R, 16), lambda i: (0, i, 0))
    return pl.pallas_call(
        _final_body,
        grid=(NPAD // _R,),
        in_specs=[
            pl.BlockSpec((1, DOUT), lambda i: (0, 0)),
            agg_spec, agg_spec, deg_spec, deg_spec,
        ],
        out_specs=pl.BlockSpec((_R, DOUT), lambda i: (i, 0)),
        out_shape=jax.ShapeDtypeStruct((NPAD, DOUT), _f32),
    )(bias, add, awd, degdd, degwd)


def _prep_edges(eidx, pad_dst):
    pad = EPAD - NE
    src = jnp.concatenate(
        [eidx[0], jnp.zeros((pad,), jnp.int32)]).reshape(NC, NS, NBLK, EB)
    dst = jnp.concatenate(
        [eidx[1], jnp.full((pad,), pad_dst, jnp.int32)]).reshape(
            NC, NS, NBLK, EB)
    offs = (jnp.arange(NCH, dtype=jnp.int32) * NPAD).reshape(NCH, 1, 1, 1, 1)
    return src[None], src[None] + offs, dst


def _chunked(h):
    hp = jnp.pad(h, ((0, NPAD - h.shape[0]), (0, 0)))
    return hp.reshape(NPAD, NCH, HC).transpose(1, 0, 2)


def kernel(feat_d, feat_w, edge_dd, edge_dw, edge_wd,
           basis0, coeff0, bias0, basis1, coeff1, bias1,
           basis2, coeff2, bias2):
    sdd1, sdd4, ddd = _prep_edges(edge_dd, ND)
    sdw1, sdw4, ddw = _prep_edges(edge_dw, NW)
    swd1, swd4, dwd = _prep_edges(edge_wd, ND)

    deg_k = _make_deg()
    degdd = deg_k(ddd)
    degwd = deg_k(dwd)
    degdw = deg_k(ddw)

    seg4 = _make_segsum(NCH, HC)
    seg1 = _make_segsum(1, DOUT)

    hd4 = _chunked(feat_d)
    hw4 = _chunked(feat_w)
    for coeff, basis, bias in ((coeff0, basis0, bias0),
                               (coeff1, basis1, bias1)):
        td = hd4.reshape(NCH * NPAD, HC)
        tw = hw4.reshape(NCH * NPAD, HC)
        add = seg4(td, sdd4, ddd)
        awd = seg4(tw, swd4, dwd)
        adw = seg4(td, sdw4, ddw)
        hd4, hw4 = _combine01(coeff, basis, bias.reshape(1, H),
                              add, awd, adw, degdd, degwd, degdw)

    pdd, pwd = _premul(coeff2, basis2, hd4, hw4)
    a2dd = seg1(pdd, sdd1, ddd)
    a2wd = seg1(pwd, swd1, dwd)
    out = _final(bias2.reshape(1, DOUT), a2dd, a2wd, degdd, degwd)
    return out[:ND]


# speedup vs baseline: 1.5090x; 1.1066x over previous
import functools

import jax
import jax.numpy as jnp
from jax import lax
from jax.experimental import pallas as pl
from jax.experimental.pallas import tpu as pltpu
from jax.experimental.pallas import tpu_sc as plsc

ND = 50000
NW = 50000
NE = 200000
H = 128
DOUT = 16
HC = 16
NCH = H // HC

NC = 2
NS = 16
NTILES = NC * NS

NPAD = 50176
EB = 128
NBLK = 49
EPT = NBLK * EB
EPAD = NTILES * EPT

ROWS_PT = NPAD // NS
ZROWS = ROWS_PT // 8
GRP = 7

_f32 = jnp.float32


def _sc_mesh():
    return plsc.VectorSubcoreMesh(core_axis_name="c", subcore_axis_name="s")


def _make_segsum(C, W):
    @functools.partial(
        pl.kernel,
        mesh=_sc_mesh(),
        compiler_params=pltpu.CompilerParams(use_tc_tiling_on_sc=False),
        out_type=jax.ShapeDtypeStruct((NC, C, NPAD, W), _f32),
        scratch_types=[
            pltpu.VMEM((NBLK, EB), jnp.int32),
            pltpu.VMEM((NBLK, EB), jnp.int32),
            pltpu.VMEM((GRP, EB, W), _f32),
            pltpu.VMEM((GRP, EB, W), _f32),
            pltpu.VMEM((ZROWS, W), _f32),
            pltpu.VMEM_SHARED((NPAD, W), _f32),
            pltpu.SemaphoreType.DMA,
            pltpu.SemaphoreType.DMA,
            pltpu.SemaphoreType.DMA,
            pltpu.SemaphoreType.DMA,
        ],
    )
    def segsum(table_hbm, sidx_hbm, didx_hbm, out_hbm,
               sidx_v, didx_v, rows_a, rows_b, zeros_v, acc_sh,
               gs_a, gs_b, ss_a, ss_b):
        cid = lax.axis_index("c")
        sid = lax.axis_index("s")

        @pl.loop(0, ZROWS)
        def _zinit(i):
            for j in range(W // 16):
                zeros_v[i, pl.ds(j * 16, 16)] = jnp.zeros((16,), _f32)

        def g_start(rows, sem, j, b):
            pltpu.async_copy(table_hbm.at[sidx_v.at[j]], rows.at[b], sem)

        def g_wait(rows, sem, b):
            pltpu.make_async_copy(
                table_hbm.at[sidx_v.at[0]], rows.at[b], sem).wait()

        def s_start(rows, sem, j, b):
            pltpu.async_copy(rows.at[b], acc_sh.at[didx_v.at[j]], sem,
                             add=True)

        def s_wait(rows, sem, b):
            pltpu.make_async_copy(
                rows.at[b], acc_sh.at[didx_v.at[0]], sem).wait()

        for c in range(C):
            for z in range(ROWS_PT // ZROWS):
                pltpu.sync_copy(
                    zeros_v,
                    acc_sh.at[pl.ds(sid * ROWS_PT + z * ZROWS, ZROWS)])
            pltpu.sync_copy(sidx_hbm.at[c, cid, sid], sidx_v)
            pltpu.sync_copy(didx_hbm.at[cid, sid], didx_v)
            plsc.subcore_barrier()

            for b in range(GRP):
                g_start(rows_a, gs_a, b, b)
            for b in range(GRP):
                g_start(rows_b, gs_b, GRP + b, b)

            @pl.loop(0, (GRP - 1) // 2)
            def _pair(t):
                for b in range(GRP):
                    g_wait(rows_a, gs_a, b)
                    s_start(rows_a, ss_a, 2 * GRP * t + b, b)
                for b in range(GRP):
                    g_wait(rows_b, gs_b, b)
                    s_start(rows_b, ss_b, 2 * GRP * t + GRP + b, b)
                for b in range(GRP):
                    s_wait(rows_a, ss_a, b)
                    g_start(rows_a, gs_a, 2 * GRP * t + 2 * GRP + b, b)

                @pl.when(t < (GRP - 1) // 2 - 1)
                def _():
                    for b in range(GRP):
                        s_wait(rows_b, ss_b, b)
                        g_start(rows_b, gs_b, 2 * GRP * t + 3 * GRP + b, b)

            for b in range(GRP):
                g_wait(rows_a, gs_a, b)
                s_start(rows_a, ss_a, (NBLK - GRP) + b, b)
            for b in range(GRP):
                s_wait(rows_b, ss_b, b)
            for b in range(GRP):
                s_wait(rows_a, ss_a, b)

            plsc.subcore_barrier()
            pltpu.sync_copy(
                acc_sh.at[pl.ds(sid * ROWS_PT, ROWS_PT)],
                out_hbm.at[cid, c, pl.ds(sid * ROWS_PT, ROWS_PT)])
            plsc.subcore_barrier()

    return segsum


def _make_deg():
    @functools.partial(
        pl.kernel,
        mesh=_sc_mesh(),
        compiler_params=pltpu.CompilerParams(use_tc_tiling_on_sc=False),
        out_type=jax.ShapeDtypeStruct((NC, NPAD, 16), _f32),
        scratch_types=[
            pltpu.VMEM((NBLK, EB), jnp.int32),
            pltpu.VMEM((EB, 16), _f32),
            pltpu.VMEM((ZROWS, 16), _f32),
            pltpu.VMEM_SHARED((NPAD, 16), _f32),
            pltpu.SemaphoreType.DMA,
        ],
    )
    def deg(didx_hbm, out_hbm, didx_v, ones_v, zeros_v, acc_sh, sem):
        cid = lax.axis_index("c")
        sid = lax.axis_index("s")

        @pl.loop(0, EB)
        def _oinit(i):
            ones_v[i, pl.ds(0, 16)] = jnp.ones((16,), _f32)

        @pl.loop(0, ZROWS)
        def _zinit(i):
            zeros_v[i, pl.ds(0, 16)] = jnp.zeros((16,), _f32)

        for z in range(ROWS_PT // ZROWS):
            pltpu.sync_copy(
                zeros_v, acc_sh.at[pl.ds(sid * ROWS_PT + z * ZROWS, ZROWS)])
        pltpu.sync_copy(didx_hbm.at[cid, sid], didx_v)
        plsc.subcore_barrier()

        def s_start(j):
            pltpu.async_copy(ones_v, acc_sh.at[didx_v.at[j]], sem, add=True)

        def s_wait():
            pltpu.make_async_copy(ones_v, acc_sh.at[didx_v.at[0]], sem).wait()

        for b in range(GRP):
            s_start(b)

        @pl.loop(1, GRP)
        def _grp(t):
            for b in range(GRP):
                s_start(t * GRP + b)
            for b in range(GRP):
                s_wait()

        for b in range(GRP):
            s_wait()

        plsc.subcore_barrier()
        pltpu.sync_copy(
            acc_sh.at[pl.ds(sid * ROWS_PT, ROWS_PT)],
            out_hbm.at[cid, pl.ds(sid * ROWS_PT, ROWS_PT)])

    return deg


_R = 256


def _mean(a_ref, d_ref):
    a = a_ref[0] + a_ref[1]
    deg = d_ref[0, :, 0:1] + d_ref[1, :, 0:1]
    r = 1.0 / jnp.maximum(deg, 1.0)
    return a * r[None]


def _combine01_body(coeff_ref, basis_ref, bias_ref,
                    add_ref, awd_ref, adw_ref,
                    ddd_ref, dwd_ref, ddw_ref,
                    outd_ref, outw_ref, w_ref):
    i = pl.program_id(0)

    @pl.when(i == 0)
    def _():
        bs = basis_ref[...]
        for r in range(3):
            w_ref[r] = coeff_ref[r, 0] * bs[0] + coeff_ref[r, 1] * bs[1]

    xdd = _mean(add_ref, ddd_ref)
    xwd = _mean(awd_ref, dwd_ref)
    xdw = _mean(adw_ref, ddw_ref)
    w = w_ref[...]
    accd = jnp.zeros((_R, H), _f32)
    accw = jnp.zeros((_R, H), _f32)
    for c in range(NCH):
        ws = w[:, c * HC:(c + 1) * HC, :]
        accd = accd + jnp.dot(xdd[c], ws[0], preferred_element_type=_f32,
                              precision=lax.Precision.HIGHEST)
        accd = accd + jnp.dot(xwd[c], ws[2], preferred_element_type=_f32,
                              precision=lax.Precision.HIGHEST)
        accw = accw + jnp.dot(xdw[c], ws[1], preferred_element_type=_f32,
                              precision=lax.Precision.HIGHEST)
    hd = jnp.maximum(accd + bias_ref[...], 0.0)
    hw = jnp.maximum(accw + bias_ref[...], 0.0)
    for c in range(NCH):
        outd_ref[c] = hd[:, c * HC:(c + 1) * HC]
        outw_ref[c] = hw[:, c * HC:(c + 1) * HC]


def _combine01(coeff, basis, bias, add, awd, adw, degdd, degwd, degdw):
    agg_spec = pl.BlockSpec((NC, NCH, _R, HC), lambda i: (0, 0, i, 0))
    deg_spec = pl.BlockSpec((NC, _R, 16), lambda i: (0, i, 0))
    out_spec = pl.BlockSpec((NCH, _R, HC), lambda i: (0, i, 0))
    return pl.pallas_call(
        _combine01_body,
        grid=(NPAD // _R,),
        in_specs=[
            pl.BlockSpec(memory_space=pltpu.SMEM),
            pl.BlockSpec((2, H, H), lambda i: (0, 0, 0)),
            pl.BlockSpec((1, H), lambda i: (0, 0)),
            agg_spec, agg_spec, agg_spec,
            deg_spec, deg_spec, deg_spec,
        ],
        out_specs=[out_spec, out_spec],
        out_shape=[
            jax.ShapeDtypeStruct((NCH, NPAD, HC), _f32),
            jax.ShapeDtypeStruct((NCH, NPAD, HC), _f32),
        ],
        scratch_shapes=[pltpu.VMEM((3, H, H), _f32)],
    )(coeff, basis, bias, add, awd, adw, degdd, degwd, degdw)


def _premul_body(coeff_ref, basis_ref, hd_ref, hw_ref,
                 pdd_ref, pwd_ref, w_ref):
    i = pl.program_id(0)

    @pl.when(i == 0)
    def _():
        bs = basis_ref[...]
        w_ref[0] = coeff_ref[0, 0] * bs[0] + coeff_ref[0, 1] * bs[1]
        w_ref[1] = coeff_ref[2, 0] * bs[0] + coeff_ref[2, 1] * bs[1]

    w = w_ref[...]
    pdd = jnp.zeros((_R, DOUT), _f32)
    pwd = jnp.zeros((_R, DOUT), _f32)
    for c in range(NCH):
        ws = w[:, c * HC:(c + 1) * HC, :]
        pdd = pdd + jnp.dot(hd_ref[c], ws[0], preferred_element_type=_f32,
                              precision=lax.Precision.HIGHEST)
        pwd = pwd + jnp.dot(hw_ref[c], ws[1], preferred_element_type=_f32,
                              precision=lax.Precision.HIGHEST)
    pdd_ref[...] = pdd
    pwd_ref[...] = pwd


def _premul(coeff, basis, hd, hw):
    h_spec = pl.BlockSpec((NCH, _R, HC), lambda i: (0, i, 0))
    out_spec = pl.BlockSpec((_R, DOUT), lambda i: (i, 0))
    return pl.pallas_call(
        _premul_body,
        grid=(NPAD // _R,),
        in_specs=[
            pl.BlockSpec(memory_space=pltpu.SMEM),
            pl.BlockSpec((2, H, DOUT), lambda i: (0, 0, 0)),
            h_spec, h_spec,
        ],
        out_specs=[out_spec, out_spec],
        out_shape=[
            jax.ShapeDtypeStruct((NPAD, DOUT), _f32),
            jax.ShapeDtypeStruct((NPAD, DOUT), _f32),
        ],
        scratch_shapes=[pltpu.VMEM((2, H, DOUT), _f32)],
    )(coeff, basis, hd, hw)


def _final_body(bias_ref, add_ref, awd_ref, ddd_ref, dwd_ref, out_ref):
    xdd = _mean(add_ref, ddd_ref)
    xwd = _mean(awd_ref, dwd_ref)
    out_ref[...] = xdd[0] + xwd[0] + bias_ref[...]


def _final(bias, add, awd, degdd, degwd):
    agg_spec = pl.BlockSpec((NC, 1, _R, DOUT), lambda i: (0, 0, i, 0))
    deg_spec = pl.BlockSpec((NC, _R, 16), lambda i: (0, i, 0))
    return pl.pallas_call(
        _final_body,
        grid=(NPAD // _R,),
        in_specs=[
            pl.BlockSpec((1, DOUT), lambda i: (0, 0)),
            agg_spec, agg_spec, deg_spec, deg_spec,
        ],
        out_specs=pl.BlockSpec((_R, DOUT), lambda i: (i, 0)),
        out_shape=jax.ShapeDtypeStruct((NPAD, DOUT), _f32),
    )(bias, add, awd, degdd, degwd)


def _prep_edges(eidx, pad_dst):
    pad = EPAD - NE
    src = jnp.concatenate(
        [eidx[0], jnp.zeros((pad,), jnp.int32)]).reshape(NC, NS, NBLK, EB)
    dst = jnp.concatenate(
        [eidx[1], jnp.full((pad,), pad_dst, jnp.int32)]).reshape(
            NC, NS, NBLK, EB)
    offs = (jnp.arange(NCH, dtype=jnp.int32) * NPAD).reshape(NCH, 1, 1, 1, 1)
    return src[None], src[None] + offs, dst


def _chunked(h):
    hp = jnp.pad(h, ((0, NPAD - h.shape[0]), (0, 0)))
    return hp.reshape(NPAD, NCH, HC).transpose(1, 0, 2)


def kernel(feat_d, feat_w, edge_dd, edge_dw, edge_wd,
           basis0, coeff0, bias0, basis1, coeff1, bias1,
           basis2, coeff2, bias2):
    sdd1, sdd4, ddd = _prep_edges(edge_dd, ND)
    sdw1, sdw4, ddw = _prep_edges(edge_dw, NW)
    swd1, swd4, dwd = _prep_edges(edge_wd, ND)

    deg_k = _make_deg()
    degdd = deg_k(ddd)
    degwd = deg_k(dwd)
    degdw = deg_k(ddw)

    seg4 = _make_segsum(NCH, HC)
    seg1 = _make_segsum(1, DOUT)

    hd4 = _chunked(feat_d)
    hw4 = _chunked(feat_w)
    for coeff, basis, bias in ((coeff0, basis0, bias0),
                               (coeff1, basis1, bias1)):
        td = hd4.reshape(NCH * NPAD, HC)
        tw = hw4.reshape(NCH * NPAD, HC)
        add = seg4(td, sdd4, ddd)
        awd = seg4(tw, swd4, dwd)
        adw = seg4(td, sdw4, ddw)
        hd4, hw4 = _combine01(coeff, basis, bias.reshape(1, H),
                              add, awd, adw, degdd, degwd, degdw)

    pdd, pwd = _premul(coeff2, basis2, hd4, hw4)
    a2dd = seg1(pdd, sdd1, ddd)
    a2wd = seg1(pwd, swd1, dwd)
    out = _final(bias2.reshape(1, DOUT), a2dd, a2wd, degdd, degwd)
    return out[:ND]


# speedup vs baseline: 2.5715x; 1.7041x over previous
import functools

import jax
import jax.numpy as jnp
from jax import lax
from jax.experimental import pallas as pl
from jax.experimental.pallas import tpu as pltpu
from jax.experimental.pallas import tpu_sc as plsc

ND = 50000
NW = 50000
NE = 200000
H = 128
DOUT = 16
HC = 16
NCH = H // HC

NC = 2
NS = 16
NTILES = NC * NS

NPAD = 50176
EB = 128
NBLK = 49
EPT = NBLK * EB
EPAD = NTILES * EPT

ROWS_PT = NPAD // NS
ZROWS = ROWS_PT // 8
GRP = 7

_f32 = jnp.float32


def _sc_mesh():
    return plsc.VectorSubcoreMesh(core_axis_name="c", subcore_axis_name="s")


def _make_segsum(C, W):
    @functools.partial(
        pl.kernel,
        mesh=_sc_mesh(),
        compiler_params=pltpu.CompilerParams(use_tc_tiling_on_sc=False),
        out_type=jax.ShapeDtypeStruct((NC, NPAD, C * W), _f32),
        scratch_types=[
            pltpu.VMEM((NBLK, EB), jnp.int32),
            pltpu.VMEM((NBLK, EB), jnp.int32),
            pltpu.VMEM((GRP, EB, W), _f32),
            pltpu.VMEM((GRP, EB, W), _f32),
            pltpu.VMEM((ZROWS, W), _f32),
            pltpu.VMEM_SHARED((NPAD, W), _f32),
            pltpu.SemaphoreType.DMA,
            pltpu.SemaphoreType.DMA,
            pltpu.SemaphoreType.DMA,
            pltpu.SemaphoreType.DMA,
        ],
    )
    def segsum(table_hbm, sidx_hbm, didx_hbm, out_hbm,
               sidx_v, didx_v, rows_a, rows_b, zeros_v, acc_sh,
               gs_a, gs_b, ss_a, ss_b):
        cid = lax.axis_index("c")
        sid = lax.axis_index("s")

        @pl.loop(0, ZROWS)
        def _zinit(i):
            for j in range(W // 16):
                zeros_v[i, pl.ds(j * 16, 16)] = jnp.zeros((16,), _f32)

        def g_start(rows, sem, j, b):
            pltpu.async_copy(table_hbm.at[sidx_v.at[j]], rows.at[b], sem)

        def g_wait(rows, sem, b):
            pltpu.make_async_copy(
                table_hbm.at[sidx_v.at[0]], rows.at[b], sem).wait()

        def s_start(rows, sem, j, b):
            pltpu.async_copy(rows.at[b], acc_sh.at[didx_v.at[j]], sem,
                             add=True)

        def s_wait(rows, sem, b):
            pltpu.make_async_copy(
                rows.at[b], acc_sh.at[didx_v.at[0]], sem).wait()

        for c in range(C):
            for z in range(ROWS_PT // ZROWS):
                pltpu.sync_copy(
                    zeros_v,
                    acc_sh.at[pl.ds(sid * ROWS_PT + z * ZROWS, ZROWS)])
            pltpu.sync_copy(sidx_hbm.at[cid, sid], sidx_v)
            pltpu.sync_copy(didx_hbm.at[cid, sid], didx_v)
            if c > 0:
                off = jnp.full((16,), c * NPAD, jnp.int32)

                @pl.loop(0, NBLK)
                def _offs(j):
                    for k in range(EB // 16):
                        sidx_v[j, pl.ds(16 * k, 16)] = (
                            sidx_v[j, pl.ds(16 * k, 16)] + off)
            plsc.subcore_barrier()

            for b in range(GRP):
                g_start(rows_a, gs_a, b, b)
            for b in range(GRP):
                g_start(rows_b, gs_b, GRP + b, b)

            @pl.loop(0, (GRP - 1) // 2)
            def _pair(t):
                for b in range(GRP):
                    g_wait(rows_a, gs_a, b)
                    s_start(rows_a, ss_a, 2 * GRP * t + b, b)
                for b in range(GRP):
                    g_wait(rows_b, gs_b, b)
                    s_start(rows_b, ss_b, 2 * GRP * t + GRP + b, b)
                for b in range(GRP):
                    s_wait(rows_a, ss_a, b)
                    g_start(rows_a, gs_a, 2 * GRP * t + 2 * GRP + b, b)

                @pl.when(t < (GRP - 1) // 2 - 1)
                def _():
                    for b in range(GRP):
                        s_wait(rows_b, ss_b, b)
                        g_start(rows_b, gs_b, 2 * GRP * t + 3 * GRP + b, b)

            for b in range(GRP):
                g_wait(rows_a, gs_a, b)
                s_start(rows_a, ss_a, (NBLK - GRP) + b, b)
            for b in range(GRP):
                s_wait(rows_b, ss_b, b)
            for b in range(GRP):
                s_wait(rows_a, ss_a, b)

            plsc.subcore_barrier()
            pltpu.sync_copy(
                acc_sh.at[pl.ds(sid * ROWS_PT, ROWS_PT)],
                out_hbm.at[cid, pl.ds(sid * ROWS_PT, ROWS_PT),
                           pl.ds(c * W, W)])
            plsc.subcore_barrier()

    return segsum


def _make_deg():
    @functools.partial(
        pl.kernel,
        mesh=_sc_mesh(),
        compiler_params=pltpu.CompilerParams(use_tc_tiling_on_sc=False),
        out_type=jax.ShapeDtypeStruct((NC, NPAD, 16), _f32),
        scratch_types=[
            pltpu.VMEM((NBLK, EB), jnp.int32),
            pltpu.VMEM((EB, 16), _f32),
            pltpu.VMEM((ZROWS, 16), _f32),
            pltpu.VMEM_SHARED((NPAD, 16), _f32),
            pltpu.SemaphoreType.DMA,
        ],
    )
    def deg(didx_hbm, out_hbm, didx_v, ones_v, zeros_v, acc_sh, sem):
        cid = lax.axis_index("c")
        sid = lax.axis_index("s")

        @pl.loop(0, EB)
        def _oinit(i):
            ones_v[i, pl.ds(0, 16)] = jnp.ones((16,), _f32)

        @pl.loop(0, ZROWS)
        def _zinit(i):
            zeros_v[i, pl.ds(0, 16)] = jnp.zeros((16,), _f32)

        for z in range(ROWS_PT // ZROWS):
            pltpu.sync_copy(
                zeros_v, acc_sh.at[pl.ds(sid * ROWS_PT + z * ZROWS, ZROWS)])
        pltpu.sync_copy(didx_hbm.at[cid, sid], didx_v)
        plsc.subcore_barrier()

        def s_start(j):
            pltpu.async_copy(ones_v, acc_sh.at[didx_v.at[j]], sem, add=True)

        def s_wait():
            pltpu.make_async_copy(ones_v, acc_sh.at[didx_v.at[0]], sem).wait()

        for b in range(GRP):
            s_start(b)

        @pl.loop(1, GRP)
        def _grp(t):
            for b in range(GRP):
                s_start(t * GRP + b)
            for b in range(GRP):
                s_wait()

        for b in range(GRP):
            s_wait()

        plsc.subcore_barrier()
        pltpu.sync_copy(
            acc_sh.at[pl.ds(sid * ROWS_PT, ROWS_PT)],
            out_hbm.at[cid, pl.ds(sid * ROWS_PT, ROWS_PT)])

    return deg


_R = 512


def _dot3(x, w):
    xh = x.astype(jnp.bfloat16)
    xl = (x - xh.astype(_f32)).astype(jnp.bfloat16)
    wh = w.astype(jnp.bfloat16)
    wl = (w - wh.astype(_f32)).astype(jnp.bfloat16)
    out = jnp.dot(xh, wh, preferred_element_type=_f32)
    out = out + jnp.dot(xh, wl, preferred_element_type=_f32)
    out = out + jnp.dot(xl, wh, preferred_element_type=_f32)
    return out


def _mean(a_ref, d_ref):
    a = a_ref[0] + a_ref[1]
    deg = d_ref[0, :, 0:1] + d_ref[1, :, 0:1]
    r = 1.0 / jnp.maximum(deg, 1.0)
    return a * r


def _mk_w(coeff_ref, basis_ref, w_ref, rows):
    bs = basis_ref[...]
    for k, r in enumerate(rows):
        w_ref[k] = coeff_ref[r, 0] * bs[0] + coeff_ref[r, 1] * bs[1]


def _hidden(w_ref, bias_ref, add_ref, awd_ref, adw_ref,
            ddd_ref, dwd_ref, ddw_ref):
    xdd = _mean(add_ref, ddd_ref)
    xwd = _mean(awd_ref, dwd_ref)
    xdw = _mean(adw_ref, ddw_ref)
    w = w_ref[...]
    hd = _dot3(xdd, w[0]) + _dot3(xwd, w[2])
    hw = _dot3(xdw, w[1])
    hd = jnp.maximum(hd + bias_ref[...], 0.0)
    hw = jnp.maximum(hw + bias_ref[...], 0.0)
    return hd, hw


def _combine0_body(coeff_ref, basis_ref, bias_ref,
                   add_ref, awd_ref, adw_ref,
                   ddd_ref, dwd_ref, ddw_ref,
                   outd_ref, outw_ref, w_ref):
    @pl.when(pl.program_id(0) == 0)
    def _():
        _mk_w(coeff_ref, basis_ref, w_ref, (0, 1, 2))

    hd, hw = _hidden(w_ref, bias_ref, add_ref, awd_ref, adw_ref,
                     ddd_ref, dwd_ref, ddw_ref)
    for c in range(NCH):
        outd_ref[c] = hd[:, c * HC:(c + 1) * HC]
        outw_ref[c] = hw[:, c * HC:(c + 1) * HC]


def _combine1_body(coeff_ref, basis_ref, bias_ref, coeff2_ref, basis2_ref,
                   add_ref, awd_ref, adw_ref,
                   ddd_ref, dwd_ref, ddw_ref,
                   pdd_ref, pwd_ref, w_ref, w2_ref):
    @pl.when(pl.program_id(0) == 0)
    def _():
        _mk_w(coeff_ref, basis_ref, w_ref, (0, 1, 2))
        _mk_w(coeff2_ref, basis2_ref, w2_ref, (0, 2))

    hd, hw = _hidden(w_ref, bias_ref, add_ref, awd_ref, adw_ref,
                     ddd_ref, dwd_ref, ddw_ref)
    w2 = w2_ref[...]
    pdd_ref[...] = _dot3(hd, w2[0])
    pwd_ref[...] = _dot3(hw, w2[1])


_agg_spec = None


def _combine_specs():
    agg_spec = pl.BlockSpec((NC, _R, H), lambda i: (0, i, 0))
    deg_spec = pl.BlockSpec((NC, _R, 16), lambda i: (0, i, 0))
    return agg_spec, deg_spec


def _combine0(coeff, basis, bias, add, awd, adw, degdd, degwd, degdw):
    agg_spec, deg_spec = _combine_specs()
    out_spec = pl.BlockSpec((NCH, _R, HC), lambda i: (0, i, 0))
    return pl.pallas_call(
        _combine0_body,
        grid=(NPAD // _R,),
        in_specs=[
            pl.BlockSpec(memory_space=pltpu.SMEM),
            pl.BlockSpec((2, H, H), lambda i: (0, 0, 0)),
            pl.BlockSpec((1, H), lambda i: (0, 0)),
            agg_spec, agg_spec, agg_spec,
            deg_spec, deg_spec, deg_spec,
        ],
        out_specs=[out_spec, out_spec],
        out_shape=[
            jax.ShapeDtypeStruct((NCH, NPAD, HC), _f32),
            jax.ShapeDtypeStruct((NCH, NPAD, HC), _f32),
        ],
        scratch_shapes=[pltpu.VMEM((3, H, H), _f32)],
    )(coeff, basis, bias, add, awd, adw, degdd, degwd, degdw)


def _combine1(coeff, basis, bias, coeff2, basis2,
              add, awd, adw, degdd, degwd, degdw):
    agg_spec, deg_spec = _combine_specs()
    out_spec = pl.BlockSpec((_R, DOUT), lambda i: (i, 0))
    return pl.pallas_call(
        _combine1_body,
        grid=(NPAD // _R,),
        in_specs=[
            pl.BlockSpec(memory_space=pltpu.SMEM),
            pl.BlockSpec((2, H, H), lambda i: (0, 0, 0)),
            pl.BlockSpec((1, H), lambda i: (0, 0)),
            pl.BlockSpec(memory_space=pltpu.SMEM),
            pl.BlockSpec((2, H, DOUT), lambda i: (0, 0, 0)),
            agg_spec, agg_spec, agg_spec,
            deg_spec, deg_spec, deg_spec,
        ],
        out_specs=[out_spec, out_spec],
        out_shape=[
            jax.ShapeDtypeStruct((NPAD, DOUT), _f32),
            jax.ShapeDtypeStruct((NPAD, DOUT), _f32),
        ],
        scratch_shapes=[pltpu.VMEM((3, H, H), _f32),
                        pltpu.VMEM((2, H, DOUT), _f32)],
    )(coeff, basis, bias, coeff2, basis2,
      add, awd, adw, degdd, degwd, degdw)


def _final_body(bias_ref, add_ref, awd_ref, ddd_ref, dwd_ref, out_ref):
    xdd = _mean(add_ref, ddd_ref)
    xwd = _mean(awd_ref, dwd_ref)
    out_ref[...] = xdd + xwd + bias_ref[...]


def _final(bias, add, awd, degdd, degwd):
    agg_spec = pl.BlockSpec((NC, _R, DOUT), lambda i: (0, i, 0))
    deg_spec = pl.BlockSpec((NC, _R, 16), lambda i: (0, i, 0))
    return pl.pallas_call(
        _final_body,
        grid=(NPAD // _R,),
        in_specs=[
            pl.BlockSpec((1, DOUT), lambda i: (0, 0)),
            agg_spec, agg_spec, deg_spec, deg_spec,
        ],
        out_specs=pl.BlockSpec((_R, DOUT), lambda i: (i, 0)),
        out_shape=jax.ShapeDtypeStruct((NPAD, DOUT), _f32),
    )(bias, add, awd, degdd, degwd)


def _prep_edges(eidx, pad_dst):
    pad = EPAD - NE
    src = jnp.concatenate(
        [eidx[0], jnp.zeros((pad,), jnp.int32)]).reshape(NC, NS, NBLK, EB)
    dst = jnp.concatenate(
        [eidx[1], jnp.full((pad,), pad_dst, jnp.int32)]).reshape(
            NC, NS, NBLK, EB)
    return src, dst


def _chunked(h):
    hp = jnp.pad(h, ((0, NPAD - h.shape[0]), (0, 0)))
    return hp.reshape(NPAD, NCH, HC).transpose(1, 0, 2)


def kernel(feat_d, feat_w, edge_dd, edge_dw, edge_wd,
           basis0, coeff0, bias0, basis1, coeff1, bias1,
           basis2, coeff2, bias2):
    sdd, ddd = _prep_edges(edge_dd, ND)
    sdw, ddw = _prep_edges(edge_dw, NW)
    swd, dwd = _prep_edges(edge_wd, ND)

    deg_k = _make_deg()
    degdd = deg_k(ddd)
    degwd = deg_k(dwd)
    degdw = deg_k(ddw)

    seg4 = _make_segsum(NCH, HC)
    seg1 = _make_segsum(1, DOUT)

    td = _chunked(feat_d).reshape(NCH * NPAD, HC)
    tw = _chunked(feat_w).reshape(NCH * NPAD, HC)

    add = seg4(td, sdd, ddd)
    awd = seg4(tw, swd, dwd)
    adw = seg4(td, sdw, ddw)
    hd4, hw4 = _combine0(coeff0, basis0, bias0.reshape(1, H),
                         add, awd, adw, degdd, degwd, degdw)

    td = hd4.reshape(NCH * NPAD, HC)
    tw = hw4.reshape(NCH * NPAD, HC)
    add = seg4(td, sdd, ddd)
    awd = seg4(tw, swd, dwd)
    adw = seg4(td, sdw, ddw)
    pdd, pwd = _combine1(coeff1, basis1, bias1.reshape(1, H),
                         coeff2, basis2,
                         add, awd, adw, degdd, degwd, degdw)

    a2dd = seg1(pdd, sdd, ddd)
    a2wd = seg1(pwd, swd, dwd)
    out = _final(bias2.reshape(1, DOUT), a2dd, a2wd, degdd, degwd)
    return out[:ND]


# speedup vs baseline: 2.6482x; 1.0298x over previous
import functools

import jax
import jax.numpy as jnp
from jax import lax
from jax.experimental import pallas as pl
from jax.experimental.pallas import tpu as pltpu
from jax.experimental.pallas import tpu_sc as plsc

ND = 50000
NW = 50000
NE = 200000
H = 128
DOUT = 16
HC = 16
NCH = H // HC

NC = 2
NS = 16
NTILES = NC * NS

NPAD = 50176
EB = 128
NBLK = 49
EPT = NBLK * EB
EPAD = NTILES * EPT

ROWS_PT = NPAD // NS
ZROWS = ROWS_PT // 8
GRP = 7

_f32 = jnp.float32


def _sc_mesh():
    return plsc.VectorSubcoreMesh(core_axis_name="c", subcore_axis_name="s")


def _make_segsum(C, W):
    @functools.partial(
        pl.kernel,
        mesh=_sc_mesh(),
        compiler_params=pltpu.CompilerParams(use_tc_tiling_on_sc=False),
        out_type=jax.ShapeDtypeStruct((NC, NPAD, C * W), _f32),
        scratch_types=[
            pltpu.VMEM((NBLK, EB), jnp.int32),
            pltpu.VMEM((NBLK, EB), jnp.int32),
            pltpu.VMEM((GRP, EB, W), _f32),
            pltpu.VMEM((GRP, EB, W), _f32),
            pltpu.VMEM((ZROWS, W), _f32),
            pltpu.VMEM_SHARED((NPAD, W), _f32),
            pltpu.SemaphoreType.DMA,
            pltpu.SemaphoreType.DMA,
            pltpu.SemaphoreType.DMA,
            pltpu.SemaphoreType.DMA,
        ],
    )
    def segsum(table_hbm, sidx_hbm, didx_hbm, out_hbm,
               sidx_v, didx_v, rows_a, rows_b, zeros_v, acc_sh,
               gs_a, gs_b, ss_a, ss_b):
        cid = lax.axis_index("c")
        sid = lax.axis_index("s")

        @pl.loop(0, ZROWS)
        def _zinit(i):
            for j in range(W // 16):
                zeros_v[i, pl.ds(j * 16, 16)] = jnp.zeros((16,), _f32)

        def g_start(rows, sem, c, j, b):
            pltpu.async_copy(
                table_hbm.at[pl.ds(c * NPAD, NPAD)].at[sidx_v.at[j]],
                rows.at[b], sem)

        def g_wait(rows, sem, c, b):
            pltpu.make_async_copy(
                table_hbm.at[pl.ds(c * NPAD, NPAD)].at[sidx_v.at[0]],
                rows.at[b], sem).wait()

        def s_start(rows, sem, j, b):
            pltpu.async_copy(rows.at[b], acc_sh.at[didx_v.at[j]], sem,
                             add=True)

        def s_wait(rows, sem, b):
            pltpu.make_async_copy(
                rows.at[b], acc_sh.at[didx_v.at[0]], sem).wait()

        pltpu.sync_copy(sidx_hbm.at[cid, sid], sidx_v)
        pltpu.sync_copy(didx_hbm.at[cid, sid], didx_v)
        for c in range(C):
            for z in range(ROWS_PT // ZROWS):
                pltpu.sync_copy(
                    zeros_v,
                    acc_sh.at[pl.ds(sid * ROWS_PT + z * ZROWS, ZROWS)])
            plsc.subcore_barrier()

            for b in range(GRP):
                g_start(rows_a, gs_a, c, b, b)
            for b in range(GRP):
                g_start(rows_b, gs_b, c, GRP + b, b)

            @pl.loop(0, (GRP - 1) // 2)
            def _pair(t):
                for b in range(GRP):
                    g_wait(rows_a, gs_a, c, b)
                    s_start(rows_a, ss_a, 2 * GRP * t + b, b)
                for b in range(GRP):
                    g_wait(rows_b, gs_b, c, b)
                    s_start(rows_b, ss_b, 2 * GRP * t + GRP + b, b)
                for b in range(GRP):
                    s_wait(rows_a, ss_a, b)
                    g_start(rows_a, gs_a, c, 2 * GRP * t + 2 * GRP + b, b)

                @pl.when(t < (GRP - 1) // 2 - 1)
                def _():
                    for b in range(GRP):
                        s_wait(rows_b, ss_b, b)
                        g_start(rows_b, gs_b, c,
                                2 * GRP * t + 3 * GRP + b, b)

            for b in range(GRP):
                g_wait(rows_a, gs_a, c, b)
                s_start(rows_a, ss_a, (NBLK - GRP) + b, b)
            for b in range(GRP):
                s_wait(rows_b, ss_b, b)
            for b in range(GRP):
                s_wait(rows_a, ss_a, b)

            plsc.subcore_barrier()
            pltpu.sync_copy(
                acc_sh.at[pl.ds(sid * ROWS_PT, ROWS_PT)],
                out_hbm.at[cid, pl.ds(sid * ROWS_PT, ROWS_PT),
                           pl.ds(c * W, W)])
            plsc.subcore_barrier()

    return segsum


def _make_deg():
    @functools.partial(
        pl.kernel,
        mesh=_sc_mesh(),
        compiler_params=pltpu.CompilerParams(use_tc_tiling_on_sc=False),
        out_type=jax.ShapeDtypeStruct((NC, NPAD, 16), _f32),
        scratch_types=[
            pltpu.VMEM((NBLK, EB), jnp.int32),
            pltpu.VMEM((EB, 16), _f32),
            pltpu.VMEM((ZROWS, 16), _f32),
            pltpu.VMEM_SHARED((NPAD, 16), _f32),
            pltpu.SemaphoreType.DMA,
        ],
    )
    def deg(didx_hbm, out_hbm, didx_v, ones_v, zeros_v, acc_sh, sem):
        cid = lax.axis_index("c")
        sid = lax.axis_index("s")

        @pl.loop(0, EB)
        def _oinit(i):
            ones_v[i, pl.ds(0, 16)] = jnp.ones((16,), _f32)

        @pl.loop(0, ZROWS)
        def _zinit(i):
            zeros_v[i, pl.ds(0, 16)] = jnp.zeros((16,), _f32)

        for z in range(ROWS_PT // ZROWS):
            pltpu.sync_copy(
                zeros_v, acc_sh.at[pl.ds(sid * ROWS_PT + z * ZROWS, ZROWS)])
        pltpu.sync_copy(didx_hbm.at[cid, sid], didx_v)
        plsc.subcore_barrier()

        def s_start(j):
            pltpu.async_copy(ones_v, acc_sh.at[didx_v.at[j]], sem, add=True)

        def s_wait():
            pltpu.make_async_copy(ones_v, acc_sh.at[didx_v.at[0]], sem).wait()

        for b in range(GRP):
            s_start(b)

        @pl.loop(1, GRP)
        def _grp(t):
            for b in range(GRP):
                s_start(t * GRP + b)
            for b in range(GRP):
                s_wait()

        for b in range(GRP):
            s_wait()

        plsc.subcore_barrier()
        pltpu.sync_copy(
            acc_sh.at[pl.ds(sid * ROWS_PT, ROWS_PT)],
            out_hbm.at[cid, pl.ds(sid * ROWS_PT, ROWS_PT)])

    return deg


_R = 512


def _dot3(x, w):
    xh = x.astype(jnp.bfloat16)
    xl = (x - xh.astype(_f32)).astype(jnp.bfloat16)
    wh = w.astype(jnp.bfloat16)
    wl = (w - wh.astype(_f32)).astype(jnp.bfloat16)
    out = jnp.dot(xh, wh, preferred_element_type=_f32)
    out = out + jnp.dot(xh, wl, preferred_element_type=_f32)
    out = out + jnp.dot(xl, wh, preferred_element_type=_f32)
    return out


def _mean(a_ref, d_ref):
    a = a_ref[0] + a_ref[1]
    deg = d_ref[0, :, 0:1] + d_ref[1, :, 0:1]
    r = 1.0 / jnp.maximum(deg, 1.0)
    return a * r


def _mk_w(coeff_ref, basis_ref, w_ref, rows):
    bs = basis_ref[...]
    for k, r in enumerate(rows):
        w_ref[k] = coeff_ref[r, 0] * bs[0] + coeff_ref[r, 1] * bs[1]


def _hidden(w_ref, bias_ref, add_ref, awd_ref, adw_ref,
            ddd_ref, dwd_ref, ddw_ref):
    xdd = _mean(add_ref, ddd_ref)
    xwd = _mean(awd_ref, dwd_ref)
    xdw = _mean(adw_ref, ddw_ref)
    w = w_ref[...]
    hd = _dot3(xdd, w[0]) + _dot3(xwd, w[2])
    hw = _dot3(xdw, w[1])
    hd = jnp.maximum(hd + bias_ref[...], 0.0)
    hw = jnp.maximum(hw + bias_ref[...], 0.0)
    return hd, hw


def _combine0_body(coeff_ref, basis_ref, bias_ref,
                   add_ref, awd_ref, adw_ref,
                   ddd_ref, dwd_ref, ddw_ref,
                   outd_ref, outw_ref, w_ref):
    @pl.when(pl.program_id(0) == 0)
    def _():
        _mk_w(coeff_ref, basis_ref, w_ref, (0, 1, 2))

    hd, hw = _hidden(w_ref, bias_ref, add_ref, awd_ref, adw_ref,
                     ddd_ref, dwd_ref, ddw_ref)
    for c in range(NCH):
        outd_ref[c] = hd[:, c * HC:(c + 1) * HC]
        outw_ref[c] = hw[:, c * HC:(c + 1) * HC]


def _combine1_body(coeff_ref, basis_ref, bias_ref, coeff2_ref, basis2_ref,
                   add_ref, awd_ref, adw_ref,
                   ddd_ref, dwd_ref, ddw_ref,
                   pdd_ref, pwd_ref, w_ref, w2_ref):
    @pl.when(pl.program_id(0) == 0)
    def _():
        _mk_w(coeff_ref, basis_ref, w_ref, (0, 1, 2))
        _mk_w(coeff2_ref, basis2_ref, w2_ref, (0, 2))

    hd, hw = _hidden(w_ref, bias_ref, add_ref, awd_ref, adw_ref,
                     ddd_ref, dwd_ref, ddw_ref)
    w2 = w2_ref[...]
    pdd_ref[...] = _dot3(hd, w2[0])
    pwd_ref[...] = _dot3(hw, w2[1])


_agg_spec = None


def _combine_specs():
    agg_spec = pl.BlockSpec((NC, _R, H), lambda i: (0, i, 0))
    deg_spec = pl.BlockSpec((NC, _R, 16), lambda i: (0, i, 0))
    return agg_spec, deg_spec


def _combine0(coeff, basis, bias, add, awd, adw, degdd, degwd, degdw):
    agg_spec, deg_spec = _combine_specs()
    out_spec = pl.BlockSpec((NCH, _R, HC), lambda i: (0, i, 0))
    return pl.pallas_call(
        _combine0_body,
        grid=(NPAD // _R,),
        in_specs=[
            pl.BlockSpec(memory_space=pltpu.SMEM),
            pl.BlockSpec((2, H, H), lambda i: (0, 0, 0)),
            pl.BlockSpec((1, H), lambda i: (0, 0)),
            agg_spec, agg_spec, agg_spec,
            deg_spec, deg_spec, deg_spec,
        ],
        out_specs=[out_spec, out_spec],
        out_shape=[
            jax.ShapeDtypeStruct((NCH, NPAD, HC), _f32),
            jax.ShapeDtypeStruct((NCH, NPAD, HC), _f32),
        ],
        scratch_shapes=[pltpu.VMEM((3, H, H), _f32)],
    )(coeff, basis, bias, add, awd, adw, degdd, degwd, degdw)


def _combine1(coeff, basis, bias, coeff2, basis2,
              add, awd, adw, degdd, degwd, degdw):
    agg_spec, deg_spec = _combine_specs()
    out_spec = pl.BlockSpec((_R, DOUT), lambda i: (i, 0))
    return pl.pallas_call(
        _combine1_body,
        grid=(NPAD // _R,),
        in_specs=[
            pl.BlockSpec(memory_space=pltpu.SMEM),
            pl.BlockSpec((2, H, H), lambda i: (0, 0, 0)),
            pl.BlockSpec((1, H), lambda i: (0, 0)),
            pl.BlockSpec(memory_space=pltpu.SMEM),
            pl.BlockSpec((2, H, DOUT), lambda i: (0, 0, 0)),
            agg_spec, agg_spec, agg_spec,
            deg_spec, deg_spec, deg_spec,
        ],
        out_specs=[out_spec, out_spec],
        out_shape=[
            jax.ShapeDtypeStruct((NPAD, DOUT), _f32),
            jax.ShapeDtypeStruct((NPAD, DOUT), _f32),
        ],
        scratch_shapes=[pltpu.VMEM((3, H, H), _f32),
                        pltpu.VMEM((2, H, DOUT), _f32)],
    )(coeff, basis, bias, coeff2, basis2,
      add, awd, adw, degdd, degwd, degdw)


def _final_body(bias_ref, add_ref, awd_ref, ddd_ref, dwd_ref, out_ref):
    xdd = _mean(add_ref, ddd_ref)
    xwd = _mean(awd_ref, dwd_ref)
    out_ref[...] = xdd + xwd + bias_ref[...]


def _final(bias, add, awd, degdd, degwd):
    agg_spec = pl.BlockSpec((NC, _R, DOUT), lambda i: (0, i, 0))
    deg_spec = pl.BlockSpec((NC, _R, 16), lambda i: (0, i, 0))
    return pl.pallas_call(
        _final_body,
        grid=(NPAD // _R,),
        in_specs=[
            pl.BlockSpec((1, DOUT), lambda i: (0, 0)),
            agg_spec, agg_spec, deg_spec, deg_spec,
        ],
        out_specs=pl.BlockSpec((_R, DOUT), lambda i: (i, 0)),
        out_shape=jax.ShapeDtypeStruct((NPAD, DOUT), _f32),
    )(bias, add, awd, degdd, degwd)


def _prep_edges(eidx, pad_dst):
    pad = EPAD - NE
    src = jnp.concatenate(
        [eidx[0], jnp.zeros((pad,), jnp.int32)]).reshape(NC, NS, NBLK, EB)
    dst = jnp.concatenate(
        [eidx[1], jnp.full((pad,), pad_dst, jnp.int32)]).reshape(
            NC, NS, NBLK, EB)
    return src, dst


def _chunked(h):
    hp = jnp.pad(h, ((0, NPAD - h.shape[0]), (0, 0)))
    return hp.reshape(NPAD, NCH, HC).transpose(1, 0, 2)


def kernel(feat_d, feat_w, edge_dd, edge_dw, edge_wd,
           basis0, coeff0, bias0, basis1, coeff1, bias1,
           basis2, coeff2, bias2):
    sdd, ddd = _prep_edges(edge_dd, ND)
    sdw, ddw = _prep_edges(edge_dw, NW)
    swd, dwd = _prep_edges(edge_wd, ND)

    deg_k = _make_deg()
    degdd = deg_k(ddd)
    degwd = deg_k(dwd)
    degdw = deg_k(ddw)

    seg4 = _make_segsum(NCH, HC)
    seg1 = _make_segsum(1, DOUT)

    td = _chunked(feat_d).reshape(NCH * NPAD, HC)
    tw = _chunked(feat_w).reshape(NCH * NPAD, HC)

    add = seg4(td, sdd, ddd)
    awd = seg4(tw, swd, dwd)
    adw = seg4(td, sdw, ddw)
    hd4, hw4 = _combine0(coeff0, basis0, bias0.reshape(1, H),
                         add, awd, adw, degdd, degwd, degdw)

    td = hd4.reshape(NCH * NPAD, HC)
    tw = hw4.reshape(NCH * NPAD, HC)
    add = seg4(td, sdd, ddd)
    awd = seg4(tw, swd, dwd)
    adw = seg4(td, sdw, ddw)
    pdd, pwd = _combine1(coeff1, basis1, bias1.reshape(1, H),
                         coeff2, basis2,
                         add, awd, adw, degdd, degwd, degdw)

    a2dd = seg1(pdd, sdd, ddd)
    a2wd = seg1(pwd, swd, dwd)
    out = _final(bias2.reshape(1, DOUT), a2dd, a2wd, degdd, degwd)
    return out[:ND]
